# Initial kernel scaffold; baseline (speedup 1.0000x reference)
import jax
import jax.numpy as jnp
from jax.experimental import pallas as pl


def kernel(item, embd_table, W, b):
    raise NotImplementedError("write your pallas kernel here")


# speedup vs baseline: 1.8836x; 1.8836x over previous
import functools

import jax
import jax.numpy as jnp
from jax import lax
from jax.experimental import pallas as pl
from jax.experimental.pallas import tpu as pltpu
from jax.experimental.pallas import tpu_sc as plsc

BATCH = 16384
EMBD_DIM = 128
GENRE_SIZE = 128
GATHER_WINDOW = 128
TC_BLOCK = 2048

_vector_mesh = plsc.VectorSubcoreMesh(
    core_axis_name="core", subcore_axis_name="subcore"
)


def _sc_gather(embd_table, item):
    idx = item.reshape(1, BATCH)

    @functools.partial(
        pl.kernel,
        out_type=jax.ShapeDtypeStruct((BATCH, EMBD_DIM), jnp.float32),
        mesh=_vector_mesh,
    )
    def gather_kernel(table_hbm, idx_hbm, out_hbm):
        def body(idx_vmem, out_vmem):
            pltpu.sync_copy(table_hbm.at[idx_vmem.at[0]], out_vmem)

        pltpu.emit_pipeline(
            body,
            grid=(BATCH // GATHER_WINDOW,),
            in_specs=[
                pl.BlockSpec((1, GATHER_WINDOW), index_map=lambda i: (0, i))
            ],
            out_specs=[
                pl.BlockSpec(
                    (GATHER_WINDOW, EMBD_DIM), index_map=lambda i: (i, 0)
                )
            ],
            core_axis_name=("core", "subcore"),
            dimension_semantics=(pltpu.PARALLEL,),
        )(idx_hbm, out_hbm)

    return gather_kernel(embd_table, idx)


def _linear_body(x_ref, w_ref, b_ref, o_ref):
    y = lax.dot_general(
        x_ref[...],
        w_ref[...],
        (((1,), (1,)), ((), ())),
        preferred_element_type=jnp.float32,
    )
    o_ref[...] = jnp.maximum(y + b_ref[...], 0.0)


def _tc_linear(x, W, b):
    b2 = b.reshape(1, GENRE_SIZE)
    return pl.pallas_call(
        _linear_body,
        grid=(BATCH // TC_BLOCK,),
        in_specs=[
            pl.BlockSpec((TC_BLOCK, EMBD_DIM), lambda i: (i, 0)),
            pl.BlockSpec((EMBD_DIM, GENRE_SIZE), lambda i: (0, 0)),
            pl.BlockSpec((1, GENRE_SIZE), lambda i: (0, 0)),
        ],
        out_specs=pl.BlockSpec((TC_BLOCK, GENRE_SIZE), lambda i: (i, 0)),
        out_shape=jax.ShapeDtypeStruct((BATCH, GENRE_SIZE), jnp.float32),
    )(x, W, b2)


def kernel(item, embd_table, W, b):
    x = _sc_gather(embd_table, item)
    return _tc_linear(x, W, b)


# speedup vs baseline: 1.9886x; 1.0557x over previous
import functools

import jax
import jax.numpy as jnp
from jax import lax
from jax.experimental import pallas as pl
from jax.experimental.pallas import tpu as pltpu
from jax.experimental.pallas import tpu_sc as plsc

BATCH = 16384
EMBD_DIM = 128
GENRE_SIZE = 128
GATHER_WINDOW = 128
TC_BLOCK = 2048

_vector_mesh = plsc.VectorSubcoreMesh(
    core_axis_name="core", subcore_axis_name="subcore"
)


N_WORKERS = 32
ROWS_PER_W = BATCH // N_WORKERS
N_WIN = ROWS_PER_W // GATHER_WINDOW


def _sc_gather(embd_table, item):
    idx2d = item.reshape(N_WORKERS * N_WIN, GATHER_WINDOW)

    @functools.partial(
        pl.kernel,
        out_type=jax.ShapeDtypeStruct((BATCH, EMBD_DIM), jnp.float32),
        mesh=_vector_mesh,
        scratch_types=[
            pltpu.VMEM((N_WIN, GATHER_WINDOW), jnp.int32),
            pltpu.VMEM((ROWS_PER_W, EMBD_DIM), jnp.float32),
            pltpu.SemaphoreType.DMA,
        ],
    )
    def gather_kernel(table_hbm, idx_hbm, out_hbm, idx_v, rows_v, sem):
        wid = lax.axis_index("subcore") * 2 + lax.axis_index("core")
        pltpu.sync_copy(idx_hbm.at[pl.ds(wid * N_WIN, N_WIN)], idx_v)
        copies = [
            pltpu.async_copy(
                table_hbm.at[idx_v.at[j]],
                rows_v.at[pl.ds(j * GATHER_WINDOW, GATHER_WINDOW)],
                sem,
            )
            for j in range(N_WIN)
        ]
        for cp in copies:
            cp.wait()
        pltpu.sync_copy(rows_v, out_hbm.at[pl.ds(wid * ROWS_PER_W, ROWS_PER_W)])

    return gather_kernel(embd_table, idx2d)


def _linear_body(x_ref, w_ref, b_ref, o_ref):
    y = lax.dot_general(
        x_ref[...],
        w_ref[...],
        (((1,), (1,)), ((), ())),
        preferred_element_type=jnp.float32,
    )
    o_ref[...] = jnp.maximum(y + b_ref[...], 0.0)


def _tc_linear(x, W, b):
    b2 = b.reshape(1, GENRE_SIZE)
    return pl.pallas_call(
        _linear_body,
        grid=(BATCH // TC_BLOCK,),
        in_specs=[
            pl.BlockSpec((TC_BLOCK, EMBD_DIM), lambda i: (i, 0)),
            pl.BlockSpec((EMBD_DIM, GENRE_SIZE), lambda i: (0, 0)),
            pl.BlockSpec((1, GENRE_SIZE), lambda i: (0, 0)),
        ],
        out_specs=pl.BlockSpec((TC_BLOCK, GENRE_SIZE), lambda i: (i, 0)),
        out_shape=jax.ShapeDtypeStruct((BATCH, GENRE_SIZE), jnp.float32),
    )(x, W, b2)


def kernel(item, embd_table, W, b):
    x = _sc_gather(embd_table, item)
    return _tc_linear(x, W, b)
